# Initial kernel scaffold; baseline (speedup 1.0000x reference)
import jax
import jax.numpy as jnp
from jax.experimental import pallas as pl


def kernel(ctx_coords, ctx_values, tgt_coords, params):
    raise NotImplementedError("write your pallas kernel here")


# speedup vs baseline: 9.4101x; 9.4101x over previous
import functools

import jax
import jax.numpy as jnp
import numpy as np
from jax.experimental import pallas as pl

_INTERPRET = False

B, NC, NT, NR, K5 = 8, 4096, 512, 784, 5
_CHIN = (4, 16, 32, 16)
_CHOUT = (16, 32, 16, 2)
_CMCO = 16


def _build_consts():
    i = np.linspace(-14.0, 14.0, 28)
    g = np.stack(np.meshgrid(i, i, indexing="ij"), -1).astype(np.float32)
    rep = g.reshape(-1, 2)
    pairs = rep[:, None, :] - rep[None, :, :]
    d2 = (pairs * pairs).sum(-1)
    idx = np.argsort(d2, axis=-1, kind="stable")[:, :K5]
    nbhd = np.stack([rep - rep[idx[:, k]] for k in range(K5)], 0)
    nbhd_flat = nbhd.reshape(K5 * NR, 2)
    G = np.zeros((K5, NR, NR), np.float32)
    for k in range(K5):
        G[k, np.arange(NR), idx[:, k]] = 1.0
    Rs, Ts = [], []
    for chin in _CHIN:
        R = np.zeros((_CMCO, _CMCO * chin), np.float32)
        T = np.zeros((chin, _CMCO * chin), np.float32)
        for m in range(_CMCO):
            for c in range(chin):
                R[m, m * chin + c] = 1.0
                T[c, m * chin + c] = 1.0
        Rs.append(R)
        Ts.append(T)
    return rep, nbhd_flat, G, Rs, Ts


_REP_NP, _NBHD_NP, _G_NP, _R_NP, _T_NP = _build_consts()


def _swish(x):
    return x * jax.nn.sigmoid(x)


def _wn_body(nbhd_ref, *refs):
    out_ref = refs[-1]
    prefs = refs[:-1]
    for l in range(4):
        x = nbhd_ref[...]
        for s in range(3):
            wt = prefs[(l * 3 + s) * 2][...]
            vec = prefs[(l * 3 + s) * 2 + 1][...]
            x = jnp.dot(x, wt, preferred_element_type=jnp.float32) + vec[0:1, :]
            mu = jnp.mean(x, axis=0, keepdims=True)
            xc = x - mu
            var = jnp.mean(xc * xc, axis=0, keepdims=True)
            x = vec[1:2, :] * xc * jax.lax.rsqrt(var + 1e-5) + vec[2:3, :]
            x = _swish(x)
        for k in range(K5):
            out_ref[l, k] = x[k * NR:(k + 1) * NR, :]


def _main_body(ctxT_ref, vals_ref, tgt_ref, rep_ref, scal_ref, g_ref, w_ref,
               wt0, wt1, wt2, wt3, bb0, bb1, bb2, bb3,
               r0, r1, r2, r3, t0, t1, t2, t3,
               res_ref, diag_ref):
    wts = (wt0, wt1, wt2, wt3)
    bbs = (bb0, bb1, bb2, bb3)
    rrs = (r0, r1, r2, r3)
    tts = (t0, t1, t2, t3)
    f32 = jnp.float32

    rep = rep_ref[...]
    ctxT = ctxT_ref[0]
    r2 = jnp.sum(rep * rep, axis=1, keepdims=True)
    c2 = jnp.sum(ctxT * ctxT, axis=0, keepdims=True)
    d = jnp.dot(rep, ctxT, preferred_element_type=f32)
    sq = jnp.maximum(r2 + c2 - 2.0 * d, 0.0)
    Km = scal_ref[0:1, 1:2] * jnp.exp(scal_ref[0:1, 0:1] * sq)
    phi = jnp.concatenate(
        [jnp.ones((NC, 1), f32), vals_ref[0]], axis=1)
    h = jnp.dot(Km, phi, preferred_element_type=f32)
    h0 = h[:, 0:1]
    h1 = h[:, 1:2] / (h0 + 1e-8)
    f = jnp.concatenate([rep, h0, h1], axis=1)

    for l in range(4):
        chin = _CHIN[l]
        partial = jnp.zeros((NR, _CMCO * chin), f32)
        for k in range(K5):
            fk = jnp.dot(g_ref[k], f, preferred_element_type=f32)
            ftile = jnp.dot(fk, tts[l][...], preferred_element_type=f32)
            wrep = jnp.dot(w_ref[l, k], rrs[l][...],
                           preferred_element_type=f32)
            partial = partial + wrep * ftile
        out = jnp.dot(partial, wts[l][...],
                      preferred_element_type=f32) + bbs[l][...]
        out = out * (1.0 / K5)
        f = _swish(out) if l < 3 else out

    tgt = tgt_ref[0]
    t2 = jnp.sum(tgt * tgt, axis=1, keepdims=True)
    repT_sq = jnp.transpose(r2)
    dr = jax.lax.dot_general(tgt, rep, (((1,), (1,)), ((), ())),
                             preferred_element_type=f32)
    sqr = jnp.maximum(t2 + repT_sq - 2.0 * dr, 0.0)
    Kr = scal_ref[0:1, 3:4] * jnp.exp(scal_ref[0:1, 2:3] * sqr)
    fcat = jnp.concatenate([f[:, 0:1], jax.nn.softplus(f[:, 1:2])], axis=1)
    res = jnp.dot(Kr, fcat, preferred_element_type=f32)
    res_ref[0] = res

    sg = res[:, 1:2]
    row = jax.lax.broadcasted_iota(jnp.int32, (NT, NT), 0)
    col = jax.lax.broadcasted_iota(jnp.int32, (NT, NT), 1)
    diag_ref[0] = jnp.where(row == col, sg, jnp.zeros((), f32))


@functools.partial(jax.jit, static_argnames=())
def kernel(ctx_coords, ctx_values, tgt_coords, params):
    f32 = jnp.float32

    nbhd = jnp.asarray(_NBHD_NP)
    wn_args = [nbhd]
    for lp in params["cnn"]:
        for sl in lp["wn"]:
            wn_args.append(jnp.transpose(sl["W"]))
            wn_args.append(jnp.stack([sl["b"], sl["g"], sl["beta"]], 0))
    w_all = pl.pallas_call(
        _wn_body,
        out_shape=jax.ShapeDtypeStruct((4, K5, NR, _CMCO), f32),
        interpret=_INTERPRET,
    )(*wn_args)

    ls1, os1 = params["psi"]["ls"], params["psi"]["os"]
    ls2, os2 = params["psi_rho"]["ls"], params["psi_rho"]["os"]
    scal = jnp.stack([-0.5 / (ls1 * ls1), os1,
                      -0.5 / (ls2 * ls2), os2]).reshape(1, 4).astype(f32)
    ctxT = jnp.transpose(ctx_coords, (0, 2, 1))
    main_args = [
        ctxT, ctx_values, tgt_coords,
        jnp.asarray(_REP_NP), scal, jnp.asarray(_G_NP), w_all,
    ]
    for l, lp in enumerate(params["cnn"]):
        main_args.append(jnp.transpose(lp["W"]))
    for l, lp in enumerate(params["cnn"]):
        main_args.append(lp["b"].reshape(1, _CHOUT[l]))
    for l in range(4):
        main_args.append(jnp.asarray(_R_NP[l]))
    for l in range(4):
        main_args.append(jnp.asarray(_T_NP[l]))

    def batched(shape):
        return pl.BlockSpec((1,) + shape, lambda b: (b,) + (0,) * len(shape))

    def full(arr):
        nd = arr.ndim
        return pl.BlockSpec(arr.shape, lambda b, _n=nd: (0,) * _n)

    in_specs = [
        batched((2, NC)), batched((NC, 1)), batched((NT, 2)),
    ] + [full(a) for a in main_args[3:]]

    res, diag = pl.pallas_call(
        _main_body,
        grid=(B,),
        in_specs=in_specs,
        out_specs=[batched((NT, 2)), batched((NT, NT))],
        out_shape=[jax.ShapeDtypeStruct((B, NT, 2), f32),
                   jax.ShapeDtypeStruct((B, NT, NT), f32)],
        interpret=_INTERPRET,
    )(*main_args)

    return res[..., 0], diag


# speedup vs baseline: 10.4493x; 1.1104x over previous
import functools

import jax
import jax.numpy as jnp
import numpy as np
from jax.experimental import pallas as pl

_INTERPRET = False

B, NC, NT, NR, K5 = 8, 4096, 512, 784, 5
_CHIN = (4, 16, 32, 16)
_CHOUT = (16, 32, 16, 2)
_CMCO = 16


def _build_consts():
    i = np.linspace(-14.0, 14.0, 28)
    g = np.stack(np.meshgrid(i, i, indexing="ij"), -1).astype(np.float32)
    rep = g.reshape(-1, 2)
    pairs = rep[:, None, :] - rep[None, :, :]
    d2 = (pairs * pairs).sum(-1)
    idx = np.argsort(d2, axis=-1, kind="stable")[:, :K5]
    nbhd = np.stack([rep - rep[idx[:, k]] for k in range(K5)], 0)
    nbhd_flat = nbhd.reshape(K5 * NR, 2)
    uniq, inv, cnt = np.unique(nbhd_flat, axis=0, return_inverse=True,
                               return_counts=True)
    U = uniq.shape[0]
    oh = np.zeros((K5 * NR, U), np.float32)
    oh[np.arange(K5 * NR), inv] = 1.0
    wv = (cnt.astype(np.float64) / (K5 * NR)).astype(np.float32).reshape(U, 1)
    G = np.zeros((K5, NR, NR), np.float32)
    for k in range(K5):
        G[k, np.arange(NR), idx[:, k]] = 1.0
    Rs, Ts = [], []
    for chin in _CHIN:
        R = np.zeros((_CMCO, _CMCO * chin), np.float32)
        T = np.zeros((chin, _CMCO * chin), np.float32)
        for m in range(_CMCO):
            for c in range(chin):
                R[m, m * chin + c] = 1.0
                T[c, m * chin + c] = 1.0
        Rs.append(R)
        Ts.append(T)
    r2 = (rep * rep).sum(-1, keepdims=True).astype(np.float32)
    lhsA = np.concatenate(
        [2.0 * rep, r2, np.ones((NR, 1), np.float32)], axis=1)
    rhsB = np.concatenate(
        [-rep.T, np.ones((1, NR), np.float32), r2.T], axis=0)
    return rep, uniq, oh, wv, G, Rs, Ts, lhsA, rhsB


(_REP_NP, _U_NP, _OH_NP, _WV_NP, _G_NP, _R_NP, _T_NP,
 _LHSA_NP, _RHSB_NP) = _build_consts()
_NU = _U_NP.shape[0]


def _swish(x):
    return x * jax.nn.sigmoid(x)


def _wn_body(u_ref, oh_ref, wv_ref, *refs):
    out_ref = refs[-1]
    prefs = refs[:-1]
    wv = wv_ref[...]
    for l in range(4):
        x = u_ref[...]
        for s in range(3):
            wt = prefs[(l * 3 + s) * 2][...]
            vec = prefs[(l * 3 + s) * 2 + 1][...]
            x = jnp.dot(x, wt, preferred_element_type=jnp.float32) + vec[0:1, :]
            mu = jnp.sum(x * wv, axis=0, keepdims=True)
            xc = x - mu
            var = jnp.sum(xc * xc * wv, axis=0, keepdims=True)
            x = vec[1:2, :] * xc * jax.lax.rsqrt(var + 1e-5) + vec[2:3, :]
            x = _swish(x)
        full = jnp.dot(oh_ref[...], x, preferred_element_type=jnp.float32)
        for k in range(K5):
            out_ref[l, k] = full[k * NR:(k + 1) * NR, :]


def _main_body(ctxT_ref, vals_ref, tgt_ref, lhsA_ref, rhsB_ref, scal_ref,
               g_ref, w_ref,
               wt0, wt1, wt2, wt3, bb0, bb1, bb2, bb3,
               r0, r1, r2, r3, t0, t1, t2, t3,
               res_ref, diag_ref):
    wts = (wt0, wt1, wt2, wt3)
    bbs = (bb0, bb1, bb2, bb3)
    rrs = (r0, r1, r2, r3)
    tts = (t0, t1, t2, t3)
    f32 = jnp.float32

    lhsA = lhsA_ref[...]
    ctxT = ctxT_ref[0]
    c2 = jnp.sum(ctxT * ctxT, axis=0, keepdims=True)
    rhsA = jnp.concatenate(
        [-ctxT, jnp.ones((1, NC), f32), c2], axis=0)
    sq = jnp.dot(lhsA, rhsA, preferred_element_type=f32)
    Km = jnp.exp(scal_ref[0:1, 0:1] * jnp.maximum(sq, 0.0))
    phi = jnp.concatenate(
        [jnp.ones((NC, 1), f32), vals_ref[0]], axis=1) * scal_ref[0:1, 1:2]
    h = jnp.dot(Km, phi, preferred_element_type=f32)
    h0 = h[:, 0:1]
    h1 = h[:, 1:2] / (h0 + 1e-8)
    rep = 0.5 * lhsA[:, 0:2]
    f = jnp.concatenate([rep, h0, h1], axis=1)

    for l in range(4):
        chin = _CHIN[l]
        partial = jnp.zeros((NR, _CMCO * chin), f32)
        for k in range(K5):
            fk = jnp.dot(g_ref[k], f, preferred_element_type=f32)
            ftile = jnp.dot(fk, tts[l][...], preferred_element_type=f32)
            wrep = jnp.dot(w_ref[l, k], rrs[l][...],
                           preferred_element_type=f32)
            partial = partial + wrep * ftile
        out = jnp.dot(partial, wts[l][...],
                      preferred_element_type=f32) + bbs[l][...]
        out = out * (1.0 / K5)
        f = _swish(out) if l < 3 else out

    tgt = tgt_ref[0]
    t2 = jnp.sum(tgt * tgt, axis=1, keepdims=True)
    lhsB = jnp.concatenate(
        [2.0 * tgt, t2, jnp.ones((NT, 1), f32)], axis=1)
    sqr = jnp.dot(lhsB, rhsB_ref[...], preferred_element_type=f32)
    Kr = jnp.exp(scal_ref[0:1, 2:3] * jnp.maximum(sqr, 0.0))
    fcat = jnp.concatenate(
        [f[:, 0:1], jax.nn.softplus(f[:, 1:2])], axis=1) * scal_ref[0:1, 3:4]
    res = jnp.dot(Kr, fcat, preferred_element_type=f32)
    res_ref[0] = res

    sg = res[:, 1:2]
    row = jax.lax.broadcasted_iota(jnp.int32, (NT, NT), 0)
    col = jax.lax.broadcasted_iota(jnp.int32, (NT, NT), 1)
    diag_ref[0] = jnp.where(row == col, sg, jnp.zeros((), f32))


@functools.partial(jax.jit, static_argnames=())
def kernel(ctx_coords, ctx_values, tgt_coords, params):
    f32 = jnp.float32

    wn_args = [jnp.asarray(_U_NP), jnp.asarray(_OH_NP), jnp.asarray(_WV_NP)]
    for lp in params["cnn"]:
        for sl in lp["wn"]:
            wn_args.append(jnp.transpose(sl["W"]))
            wn_args.append(jnp.stack([sl["b"], sl["g"], sl["beta"]], 0))
    w_all = pl.pallas_call(
        _wn_body,
        out_shape=jax.ShapeDtypeStruct((4, K5, NR, _CMCO), f32),
        interpret=_INTERPRET,
    )(*wn_args)

    ls1, os1 = params["psi"]["ls"], params["psi"]["os"]
    ls2, os2 = params["psi_rho"]["ls"], params["psi_rho"]["os"]
    scal = jnp.stack([-0.5 / (ls1 * ls1), os1,
                      -0.5 / (ls2 * ls2), os2]).reshape(1, 4).astype(f32)
    ctxT = jnp.transpose(ctx_coords, (0, 2, 1))
    main_args = [
        ctxT, ctx_values, tgt_coords,
        jnp.asarray(_LHSA_NP), jnp.asarray(_RHSB_NP), scal,
        jnp.asarray(_G_NP), w_all,
    ]
    for lp in params["cnn"]:
        main_args.append(jnp.transpose(lp["W"]))
    for l, lp in enumerate(params["cnn"]):
        main_args.append(lp["b"].reshape(1, _CHOUT[l]))
    for l in range(4):
        main_args.append(jnp.asarray(_R_NP[l]))
    for l in range(4):
        main_args.append(jnp.asarray(_T_NP[l]))

    def batched(shape):
        return pl.BlockSpec((1,) + shape, lambda b: (b,) + (0,) * len(shape))

    def full(arr):
        nd = arr.ndim
        return pl.BlockSpec(arr.shape, lambda b, _n=nd: (0,) * _n)

    in_specs = [
        batched((2, NC)), batched((NC, 1)), batched((NT, 2)),
    ] + [full(a) for a in main_args[3:]]

    res, diag = pl.pallas_call(
        _main_body,
        grid=(B,),
        in_specs=in_specs,
        out_specs=[batched((NT, 2)), batched((NT, NT))],
        out_shape=[jax.ShapeDtypeStruct((B, NT, 2), f32),
                   jax.ShapeDtypeStruct((B, NT, NT), f32)],
        interpret=_INTERPRET,
    )(*main_args)

    return res[..., 0], diag


# speedup vs baseline: 14.9766x; 1.4333x over previous
import functools

import jax
import jax.numpy as jnp
import numpy as np
from jax.experimental import pallas as pl

_INTERPRET = False

B, NC, NT, NR, K5 = 8, 4096, 512, 784, 5
_CHIN = (4, 16, 32, 16)
_CHOUT = (16, 32, 16, 2)
_CMCO = 16


def _build_consts():
    i = np.linspace(-14.0, 14.0, 28)
    g = np.stack(np.meshgrid(i, i, indexing="ij"), -1).astype(np.float32)
    rep = g.reshape(-1, 2)
    pairs = rep[:, None, :] - rep[None, :, :]
    d2 = (pairs * pairs).sum(-1)
    idx = np.argsort(d2, axis=-1, kind="stable")[:, :K5]
    nbhd = np.stack([rep - rep[idx[:, k]] for k in range(K5)], 0)
    nbhd_flat = nbhd.reshape(K5 * NR, 2)
    uniq, inv, cnt = np.unique(nbhd_flat, axis=0, return_inverse=True,
                               return_counts=True)
    U = uniq.shape[0]
    oh = np.zeros((K5 * NR, U), np.float32)
    oh[np.arange(K5 * NR), inv] = 1.0
    wv = (cnt.astype(np.float64) / (K5 * NR)).astype(np.float32).reshape(U, 1)
    G = np.zeros((K5 - 1, 7, 128, 256), np.float32)
    for k in range(1, K5):
        for n in range(NR):
            t, r = divmod(n, 128)
            G[k - 1, t, r, idx[n, k] - 128 * t + 64] = 1.0
    Rs, Ts = [], []
    for chin in _CHIN:
        R = np.zeros((_CMCO, _CMCO * chin), np.float32)
        T = np.zeros((chin, _CMCO * chin), np.float32)
        for m in range(_CMCO):
            for c in range(chin):
                R[m, m * chin + c] = 1.0
                T[c, m * chin + c] = 1.0
        Rs.append(R)
        Ts.append(T)
    r2 = (rep * rep).sum(-1, keepdims=True).astype(np.float32)
    lhsA = np.concatenate(
        [2.0 * rep, r2, np.ones((NR, 1), np.float32)], axis=1)
    rhsB = np.concatenate(
        [-rep.T, np.ones((1, NR), np.float32), r2.T], axis=0)
    return rep, uniq, oh, wv, G, Rs, Ts, lhsA, rhsB


(_REP_NP, _U_NP, _OH_NP, _WV_NP, _G_NP, _R_NP, _T_NP,
 _LHSA_NP, _RHSB_NP) = _build_consts()
_NU = _U_NP.shape[0]


def _swish(x):
    return x * jax.nn.sigmoid(x)


def _wn_body(u_ref, oh_ref, wv_ref, *refs):
    out_ref = refs[-1]
    prefs = refs[:-1]
    wv = wv_ref[...]
    for l in range(4):
        x = u_ref[...]
        for s in range(3):
            wt = prefs[(l * 3 + s) * 2][...]
            vec = prefs[(l * 3 + s) * 2 + 1][...]
            x = jnp.dot(x, wt, preferred_element_type=jnp.float32) + vec[0:1, :]
            mu = jnp.sum(x * wv, axis=0, keepdims=True)
            xc = x - mu
            var = jnp.sum(xc * xc * wv, axis=0, keepdims=True)
            x = vec[1:2, :] * xc * jax.lax.rsqrt(var + 1e-5) + vec[2:3, :]
            x = _swish(x)
        full = jnp.dot(oh_ref[...], x, preferred_element_type=jnp.float32)
        for k in range(K5):
            out_ref[l, k] = full[k * NR:(k + 1) * NR, :]


def _main_body(ctxT_ref, vals_ref, tgt_ref, lhsA_ref, rhsB_ref, scal_ref,
               g_ref, w_ref,
               wt0, wt1, wt2, wt3, bb0, bb1, bb2, bb3,
               r0, r1, r2, r3, t0, t1, t2, t3,
               res_ref, diag_ref):
    wts = (wt0, wt1, wt2, wt3)
    bbs = (bb0, bb1, bb2, bb3)
    rrs = (r0, r1, r2, r3)
    tts = (t0, t1, t2, t3)
    f32 = jnp.float32

    lhsA = lhsA_ref[...]
    ctxT = ctxT_ref[0]
    c2 = jnp.sum(ctxT * ctxT, axis=0, keepdims=True)
    rhsA = jnp.concatenate(
        [-ctxT, jnp.ones((1, NC), f32), c2], axis=0)
    sq = jnp.dot(lhsA, rhsA, preferred_element_type=f32)
    Km = jnp.exp(scal_ref[0:1, 0:1] * jnp.maximum(sq, 0.0))
    phi = jnp.concatenate(
        [jnp.ones((NC, 1), f32), vals_ref[0]], axis=1) * scal_ref[0:1, 1:2]
    h = jnp.dot(Km, phi, preferred_element_type=f32)
    h0 = h[:, 0:1]
    h1 = h[:, 1:2] / (h0 + 1e-8)
    rep = 0.5 * lhsA[:, 0:2]
    f = jnp.concatenate([rep, h0, h1], axis=1)

    for l in range(4):
        chin = _CHIN[l]
        fpad = jnp.concatenate(
            [jnp.zeros((64, chin), f32), f, jnp.zeros((176, chin), f32)], 0)
        partial = jnp.zeros((NR, _CMCO * chin), f32)
        for k in range(K5):
            if k == 0:
                fk = f
            else:
                fk = jnp.concatenate(
                    [jnp.dot(g_ref[k - 1, t],
                             fpad[128 * t:128 * t + 256],
                             preferred_element_type=f32)
                     for t in range(7)], axis=0)[:NR]
            ftile = jnp.dot(fk, tts[l][...], preferred_element_type=f32)
            wrep = jnp.dot(w_ref[l, k], rrs[l][...],
                           preferred_element_type=f32)
            partial = partial + wrep * ftile
        out = jnp.dot(partial, wts[l][...],
                      preferred_element_type=f32) + bbs[l][...]
        out = out * (1.0 / K5)
        f = _swish(out) if l < 3 else out

    tgt = tgt_ref[0]
    t2 = jnp.sum(tgt * tgt, axis=1, keepdims=True)
    lhsB = jnp.concatenate(
        [2.0 * tgt, t2, jnp.ones((NT, 1), f32)], axis=1)
    sqr = jnp.dot(lhsB, rhsB_ref[...], preferred_element_type=f32)
    Kr = jnp.exp(scal_ref[0:1, 2:3] * jnp.maximum(sqr, 0.0))
    fcat = jnp.concatenate(
        [f[:, 0:1], jax.nn.softplus(f[:, 1:2])], axis=1) * scal_ref[0:1, 3:4]
    res = jnp.dot(Kr, fcat, preferred_element_type=f32)
    res_ref[0] = res

    sg = res[:, 1:2]
    row = jax.lax.broadcasted_iota(jnp.int32, (NT, NT), 0)
    col = jax.lax.broadcasted_iota(jnp.int32, (NT, NT), 1)
    diag_ref[0] = jnp.where(row == col, sg, jnp.zeros((), f32))


@functools.partial(jax.jit, static_argnames=())
def kernel(ctx_coords, ctx_values, tgt_coords, params):
    f32 = jnp.float32

    wn_args = [jnp.asarray(_U_NP), jnp.asarray(_OH_NP), jnp.asarray(_WV_NP)]
    for lp in params["cnn"]:
        for sl in lp["wn"]:
            wn_args.append(jnp.transpose(sl["W"]))
            wn_args.append(jnp.stack([sl["b"], sl["g"], sl["beta"]], 0))
    w_all = pl.pallas_call(
        _wn_body,
        out_shape=jax.ShapeDtypeStruct((4, K5, NR, _CMCO), f32),
        interpret=_INTERPRET,
    )(*wn_args)

    ls1, os1 = params["psi"]["ls"], params["psi"]["os"]
    ls2, os2 = params["psi_rho"]["ls"], params["psi_rho"]["os"]
    scal = jnp.stack([-0.5 / (ls1 * ls1), os1,
                      -0.5 / (ls2 * ls2), os2]).reshape(1, 4).astype(f32)
    ctxT = jnp.transpose(ctx_coords, (0, 2, 1))
    main_args = [
        ctxT, ctx_values, tgt_coords,
        jnp.asarray(_LHSA_NP), jnp.asarray(_RHSB_NP), scal,
        jnp.asarray(_G_NP), w_all,
    ]
    for lp in params["cnn"]:
        main_args.append(jnp.transpose(lp["W"]))
    for l, lp in enumerate(params["cnn"]):
        main_args.append(lp["b"].reshape(1, _CHOUT[l]))
    for l in range(4):
        main_args.append(jnp.asarray(_R_NP[l]))
    for l in range(4):
        main_args.append(jnp.asarray(_T_NP[l]))

    def batched(shape):
        return pl.BlockSpec((1,) + shape, lambda b: (b,) + (0,) * len(shape))

    def full(arr):
        nd = arr.ndim
        return pl.BlockSpec(arr.shape, lambda b, _n=nd: (0,) * _n)

    in_specs = [
        batched((2, NC)), batched((NC, 1)), batched((NT, 2)),
    ] + [full(a) for a in main_args[3:]]

    res, diag = pl.pallas_call(
        _main_body,
        grid=(B,),
        in_specs=in_specs,
        out_specs=[batched((NT, 2)), batched((NT, NT))],
        out_shape=[jax.ShapeDtypeStruct((B, NT, 2), f32),
                   jax.ShapeDtypeStruct((B, NT, NT), f32)],
        interpret=_INTERPRET,
    )(*main_args)

    return res[..., 0], diag


# speedup vs baseline: 16.3618x; 1.0925x over previous
import functools

import jax
import jax.numpy as jnp
import numpy as np
from jax.experimental import pallas as pl

_INTERPRET = False

B, NC, NT, NR, K5 = 8, 4096, 512, 784, 5
_CHIN = (4, 16, 32, 16)
_CHOUT = (16, 32, 16, 2)
_CMCO = 16


def _build_consts():
    i = np.linspace(-14.0, 14.0, 28)
    g = np.stack(np.meshgrid(i, i, indexing="ij"), -1).astype(np.float32)
    rep = g.reshape(-1, 2)
    pairs = rep[:, None, :] - rep[None, :, :]
    d2 = (pairs * pairs).sum(-1)
    idx = np.argsort(d2, axis=-1, kind="stable")[:, :K5]
    nbhd = np.stack([rep - rep[idx[:, k]] for k in range(K5)], 0)
    nbhd_flat = nbhd.reshape(K5 * NR, 2)
    uniq, inv, cnt = np.unique(nbhd_flat, axis=0, return_inverse=True,
                               return_counts=True)
    U = uniq.shape[0]
    oh = np.zeros((K5 * NR, U), np.float32)
    oh[np.arange(K5 * NR), inv] = 1.0
    wv = (cnt.astype(np.float64) / (K5 * NR)).astype(np.float32).reshape(U, 1)
    G = np.zeros((K5 - 1, 7, 128, 256), np.float32)
    for k in range(1, K5):
        for n in range(NR):
            t, r = divmod(n, 128)
            G[k - 1, t, r, idx[n, k] - 128 * t + 64] = 1.0
    Rs, Ts = [], []
    for l, chin in enumerate(_CHIN):
        ce = chin if l < 2 else _CHOUT[l]
        R = np.zeros((_CMCO, _CMCO * ce), np.float32)
        for m in range(_CMCO):
            R[m, m * ce:(m + 1) * ce] = 1.0
        Rs.append(R)
        if l < 2:
            T = np.zeros((chin, _CMCO * chin), np.float32)
            for m in range(_CMCO):
                T[:, m * chin:(m + 1) * chin] = np.eye(chin)
        else:
            T = np.tile(np.eye(ce, dtype=np.float32), (_CMCO, 1))
        Ts.append(T)
    r2 = (rep * rep).sum(-1, keepdims=True).astype(np.float32)
    lhsA = np.concatenate(
        [2.0 * rep, r2, np.ones((NR, 1), np.float32)], axis=1)
    rhsB = np.concatenate(
        [-rep.T, np.ones((1, NR), np.float32), r2.T], axis=0)
    return rep, uniq, oh, wv, G, Rs, Ts, lhsA, rhsB


(_REP_NP, _U_NP, _OH_NP, _WV_NP, _G_NP, _R_NP, _T_NP,
 _LHSA_NP, _RHSB_NP) = _build_consts()
_NU = _U_NP.shape[0]


def _swish(x):
    return x * jax.nn.sigmoid(x)


def _wn_body(u_ref, oh_ref, wv_ref, *refs):
    out_ref = refs[-1]
    prefs = refs[:-1]
    wv = wv_ref[...]
    for l in range(4):
        x = u_ref[...]
        for s in range(3):
            wt = prefs[(l * 3 + s) * 2][...]
            vec = prefs[(l * 3 + s) * 2 + 1][...]
            x = jnp.dot(x, wt, preferred_element_type=jnp.float32) + vec[0:1, :]
            mu = jnp.sum(x * wv, axis=0, keepdims=True)
            xc = x - mu
            var = jnp.sum(xc * xc * wv, axis=0, keepdims=True)
            x = vec[1:2, :] * xc * jax.lax.rsqrt(var + 1e-5) + vec[2:3, :]
            x = _swish(x)
        full = jnp.dot(oh_ref[...], x, preferred_element_type=jnp.float32)
        for k in range(K5):
            out_ref[l, k] = full[k * NR:(k + 1) * NR, :]


def _main_body(ctxT_ref, vals_ref, tgt_ref, lhsA_ref, rhsB_ref, scal_ref,
               g_ref, w_ref,
               wt0, wt1, wt2, wt3, bb0, bb1, bb2, bb3,
               r0, r1, r2, r3, t0, t1, t2, t3,
               res_ref, diag_ref):
    wts = (wt0, wt1, wt2, wt3)
    bbs = (bb0, bb1, bb2, bb3)
    rrs = (r0, r1, r2, r3)
    tts = (t0, t1, t2, t3)
    f32 = jnp.float32

    lhsA = lhsA_ref[...]
    ctxT = ctxT_ref[0]
    c2 = jnp.sum(ctxT * ctxT, axis=0, keepdims=True)
    rhsA = jnp.concatenate(
        [-ctxT, jnp.ones((1, NC), f32), c2], axis=0)
    sq = jnp.dot(lhsA, rhsA, preferred_element_type=f32)
    Km = jnp.exp(scal_ref[0:1, 0:1] * jnp.maximum(sq, 0.0))
    phi = jnp.concatenate(
        [jnp.ones((NC, 1), f32), vals_ref[0]], axis=1) * scal_ref[0:1, 1:2]
    h = jnp.dot(Km, phi, preferred_element_type=f32)
    h0 = h[:, 0:1]
    h1 = h[:, 1:2] / (h0 + 1e-8)
    rep = 0.5 * lhsA[:, 0:2]
    f = jnp.concatenate([rep, h0, h1], axis=1)

    for l in range(4):
        chin = _CHIN[l]
        ce = chin if l < 2 else _CHOUT[l]
        fpad = jnp.concatenate(
            [jnp.zeros((64, chin), f32), f, jnp.zeros((176, chin), f32)], 0)
        partial = jnp.zeros((NR, _CMCO * ce), f32)
        for k in range(K5):
            if k == 0:
                fk = f
            else:
                fk = jnp.concatenate(
                    [jnp.dot(g_ref[k - 1, t],
                             fpad[128 * t:128 * t + 256],
                             preferred_element_type=f32)
                     for t in range(7)], axis=0)[:NR]
            ftile = jnp.dot(fk, (tts[l] if l < 2 else wts[l])[...],
                            preferred_element_type=f32)
            wrep = jnp.dot(w_ref[l, k], rrs[l][...],
                           preferred_element_type=f32)
            partial = partial + wrep * ftile
        out = jnp.dot(partial, (wts[l] if l < 2 else tts[l])[...],
                      preferred_element_type=f32) + bbs[l][...]
        out = out * (1.0 / K5)
        f = _swish(out) if l < 3 else out

    tgt = tgt_ref[0]
    t2 = jnp.sum(tgt * tgt, axis=1, keepdims=True)
    lhsB = jnp.concatenate(
        [2.0 * tgt, t2, jnp.ones((NT, 1), f32)], axis=1)
    sqr = jnp.dot(lhsB, rhsB_ref[...], preferred_element_type=f32)
    Kr = jnp.exp(scal_ref[0:1, 2:3] * jnp.maximum(sqr, 0.0))
    fcat = jnp.concatenate(
        [f[:, 0:1], jax.nn.softplus(f[:, 1:2])], axis=1) * scal_ref[0:1, 3:4]
    res = jnp.dot(Kr, fcat, preferred_element_type=f32)
    res_ref[0] = res

    sg = res[:, 1:2]
    row = jax.lax.broadcasted_iota(jnp.int32, (NT, NT), 0)
    col = jax.lax.broadcasted_iota(jnp.int32, (NT, NT), 1)
    diag_ref[0] = jnp.where(row == col, sg, jnp.zeros((), f32))


@functools.partial(jax.jit, static_argnames=())
def kernel(ctx_coords, ctx_values, tgt_coords, params):
    f32 = jnp.float32

    wn_args = [jnp.asarray(_U_NP), jnp.asarray(_OH_NP), jnp.asarray(_WV_NP)]
    for lp in params["cnn"]:
        for sl in lp["wn"]:
            wn_args.append(jnp.transpose(sl["W"]))
            wn_args.append(jnp.stack([sl["b"], sl["g"], sl["beta"]], 0))
    w_all = pl.pallas_call(
        _wn_body,
        out_shape=jax.ShapeDtypeStruct((4, K5, NR, _CMCO), f32),
        interpret=_INTERPRET,
    )(*wn_args)

    ls1, os1 = params["psi"]["ls"], params["psi"]["os"]
    ls2, os2 = params["psi_rho"]["ls"], params["psi_rho"]["os"]
    scal = jnp.stack([-0.5 / (ls1 * ls1), os1,
                      -0.5 / (ls2 * ls2), os2]).reshape(1, 4).astype(f32)
    ctxT = jnp.transpose(ctx_coords, (0, 2, 1))
    main_args = [
        ctxT, ctx_values, tgt_coords,
        jnp.asarray(_LHSA_NP), jnp.asarray(_RHSB_NP), scal,
        jnp.asarray(_G_NP), w_all,
    ]
    for l, lp in enumerate(params["cnn"]):
        if l < 2:
            main_args.append(jnp.transpose(lp["W"]))
        else:
            co, ci = _CHOUT[l], _CHIN[l]
            w2 = jnp.transpose(lp["W"].reshape(co, _CMCO, ci), (2, 1, 0))
            main_args.append(w2.reshape(ci, _CMCO * co))
    for l, lp in enumerate(params["cnn"]):
        main_args.append(lp["b"].reshape(1, _CHOUT[l]))
    for l in range(4):
        main_args.append(jnp.asarray(_R_NP[l]))
    for l in range(4):
        main_args.append(jnp.asarray(_T_NP[l]))

    def batched(shape):
        return pl.BlockSpec((1,) + shape, lambda b: (b,) + (0,) * len(shape))

    def full(arr):
        nd = arr.ndim
        return pl.BlockSpec(arr.shape, lambda b, _n=nd: (0,) * _n)

    in_specs = [
        batched((2, NC)), batched((NC, 1)), batched((NT, 2)),
    ] + [full(a) for a in main_args[3:]]

    res, diag = pl.pallas_call(
        _main_body,
        grid=(B,),
        in_specs=in_specs,
        out_specs=[batched((NT, 2)), batched((NT, NT))],
        out_shape=[jax.ShapeDtypeStruct((B, NT, 2), f32),
                   jax.ShapeDtypeStruct((B, NT, NT), f32)],
        interpret=_INTERPRET,
    )(*main_args)

    return res[..., 0], diag


# speedup vs baseline: 17.1144x; 1.0460x over previous
import functools

import jax
import jax.numpy as jnp
import numpy as np
from jax.experimental import pallas as pl

_INTERPRET = False

B, NC, NT, NR, K5 = 8, 4096, 512, 784, 5
_CHIN = (4, 16, 32, 16)
_CHOUT = (16, 32, 16, 2)
_CMCO = 16


def _build_consts():
    i = np.linspace(-14.0, 14.0, 28)
    g = np.stack(np.meshgrid(i, i, indexing="ij"), -1).astype(np.float32)
    rep = g.reshape(-1, 2)
    pairs = rep[:, None, :] - rep[None, :, :]
    d2 = (pairs * pairs).sum(-1)
    idx = np.argsort(d2, axis=-1, kind="stable")[:, :K5]
    nbhd = np.stack([rep - rep[idx[:, k]] for k in range(K5)], 0)
    nbhd_flat = nbhd.reshape(K5 * NR, 2)
    uniq, inv, cnt = np.unique(nbhd_flat, axis=0, return_inverse=True,
                               return_counts=True)
    U = uniq.shape[0]
    oh = np.zeros((K5 * NR, U), np.float32)
    oh[np.arange(K5 * NR), inv] = 1.0
    wv = (cnt.astype(np.float64) / (K5 * NR)).astype(np.float32).reshape(U, 1)
    G = np.zeros((K5 - 1, 7, 128, 256), np.float32)
    for k in range(1, K5):
        for n in range(NR):
            t, r = divmod(n, 128)
            G[k - 1, t, r, idx[n, k] - 128 * t + 64] = 1.0
    Rs, Ts = [], []
    for l, chin in enumerate(_CHIN):
        ce = chin if l < 2 else _CHOUT[l]
        R = np.zeros((_CMCO, _CMCO * ce), np.float32)
        for m in range(_CMCO):
            R[m, m * ce:(m + 1) * ce] = 1.0
        Rs.append(R)
        if l < 2:
            T = np.zeros((chin, _CMCO * chin), np.float32)
            for m in range(_CMCO):
                T[:, m * chin:(m + 1) * chin] = np.eye(chin)
        else:
            T = np.tile(np.eye(ce, dtype=np.float32), (_CMCO, 1))
        Ts.append(T)
    r2 = (rep * rep).sum(-1, keepdims=True).astype(np.float32)
    lhsA = np.concatenate(
        [2.0 * rep, r2, np.ones((NR, 1), np.float32)], axis=1)
    rhsB = np.concatenate(
        [-rep.T, np.ones((1, NR), np.float32), r2.T], axis=0)
    return rep, uniq, oh, wv, G, Rs, Ts, lhsA, rhsB


(_REP_NP, _U_NP, _OH_NP, _WV_NP, _G_NP, _R_NP, _T_NP,
 _LHSA_NP, _RHSB_NP) = _build_consts()
_NU = _U_NP.shape[0]


def _swish(x):
    return x * jax.nn.sigmoid(x)


def _wn_body(u_ref, oh_ref, wv_ref, r0, r1, r2, r3, *refs):
    outs = refs[-4:]
    prefs = refs[:-4]
    rrs = (r0, r1, r2, r3)
    wv = wv_ref[...]
    for l in range(4):
        x = u_ref[...]
        for s in range(3):
            wt = prefs[(l * 3 + s) * 2][...]
            vec = prefs[(l * 3 + s) * 2 + 1][...]
            x = jnp.dot(x, wt, preferred_element_type=jnp.float32) + vec[0:1, :]
            mu = jnp.sum(x * wv, axis=0, keepdims=True)
            xc = x - mu
            var = jnp.sum(xc * xc * wv, axis=0, keepdims=True)
            x = vec[1:2, :] * xc * jax.lax.rsqrt(var + 1e-5) + vec[2:3, :]
            x = _swish(x)
        full = jnp.dot(oh_ref[...], x, preferred_element_type=jnp.float32)
        wexp = jnp.dot(full, rrs[l][...], preferred_element_type=jnp.float32)
        for k in range(K5):
            outs[l][k] = wexp[k * NR:(k + 1) * NR, :]


def _main_body(ctxT_ref, vals_ref, tgt_ref, lhsA_ref, rhsB_ref, scal_ref,
               g_ref, w0_ref, w1_ref, w2_ref, w3_ref,
               wt0, wt1, wt2, wt3, bb0, bb1, bb2, bb3,
               t0, t1, t2, t3,
               res_ref, diag_ref):
    wts = (wt0, wt1, wt2, wt3)
    bbs = (bb0, bb1, bb2, bb3)
    wxs = (w0_ref, w1_ref, w2_ref, w3_ref)
    tts = (t0, t1, t2, t3)
    f32 = jnp.float32

    lhsA = lhsA_ref[...]
    ctxT = ctxT_ref[0]
    c2 = jnp.sum(ctxT * ctxT, axis=0, keepdims=True)
    rhsA = jnp.concatenate(
        [-ctxT, jnp.ones((1, NC), f32), c2], axis=0)
    sq = jnp.dot(lhsA, rhsA, preferred_element_type=f32)
    Km = jnp.exp(scal_ref[0:1, 0:1] * jnp.maximum(sq, 0.0))
    phi = jnp.concatenate(
        [jnp.ones((NC, 1), f32), vals_ref[0]], axis=1) * scal_ref[0:1, 1:2]
    h = jnp.dot(Km, phi, preferred_element_type=f32)
    h0 = h[:, 0:1]
    h1 = h[:, 1:2] / (h0 + 1e-8)
    rep = 0.5 * lhsA[:, 0:2]
    f = jnp.concatenate([rep, h0, h1], axis=1)

    for l in range(4):
        chin = _CHIN[l]
        ce = chin if l < 2 else _CHOUT[l]
        fpad = jnp.concatenate(
            [jnp.zeros((64, chin), f32), f, jnp.zeros((176, chin), f32)], 0)
        partial = jnp.zeros((NR, _CMCO * ce), f32)
        for k in range(K5):
            if k == 0:
                fk = f
            else:
                fk = jnp.concatenate(
                    [jnp.dot(g_ref[k - 1, t],
                             fpad[128 * t:128 * t + 256],
                             preferred_element_type=f32)
                     for t in range(7)], axis=0)[:NR]
            ftile = jnp.dot(fk, (tts[l] if l < 2 else wts[l])[...],
                            preferred_element_type=f32)
            partial = partial + wxs[l][k] * ftile
        out = jnp.dot(partial, (wts[l] if l < 2 else tts[l])[...],
                      preferred_element_type=f32) + bbs[l][...]
        f = _swish(out) if l < 3 else out

    tgt = tgt_ref[0]
    t2 = jnp.sum(tgt * tgt, axis=1, keepdims=True)
    lhsB = jnp.concatenate(
        [2.0 * tgt, t2, jnp.ones((NT, 1), f32)], axis=1)
    sqr = jnp.dot(lhsB, rhsB_ref[...], preferred_element_type=f32)
    Kr = jnp.exp(scal_ref[0:1, 2:3] * jnp.maximum(sqr, 0.0))
    fcat = jnp.concatenate(
        [f[:, 0:1], jax.nn.softplus(f[:, 1:2])], axis=1) * scal_ref[0:1, 3:4]
    res = jnp.dot(Kr, fcat, preferred_element_type=f32)
    res_ref[0] = res

    sg = res[:, 1:2]
    row = jax.lax.broadcasted_iota(jnp.int32, (NT, NT), 0)
    col = jax.lax.broadcasted_iota(jnp.int32, (NT, NT), 1)
    diag_ref[0] = jnp.where(row == col, sg, jnp.zeros((), f32))


@functools.partial(jax.jit, static_argnames=())
def kernel(ctx_coords, ctx_values, tgt_coords, params):
    f32 = jnp.float32

    wn_args = [jnp.asarray(_U_NP), jnp.asarray(_OH_NP), jnp.asarray(_WV_NP)]
    wn_args += [jnp.asarray(_R_NP[l]) for l in range(4)]
    for lp in params["cnn"]:
        for sl in lp["wn"]:
            wn_args.append(jnp.transpose(sl["W"]))
            wn_args.append(jnp.stack([sl["b"], sl["g"], sl["beta"]], 0))
    _ce = [_CHIN[0], _CHIN[1], _CHOUT[2], _CHOUT[3]]
    w_exp = pl.pallas_call(
        _wn_body,
        out_shape=[jax.ShapeDtypeStruct((K5, NR, _CMCO * _ce[l]), f32)
                   for l in range(4)],
        interpret=_INTERPRET,
    )(*wn_args)

    ls1, os1 = params["psi"]["ls"], params["psi"]["os"]
    ls2, os2 = params["psi_rho"]["ls"], params["psi_rho"]["os"]
    scal = jnp.stack([-0.5 / (ls1 * ls1), os1,
                      -0.5 / (ls2 * ls2), os2]).reshape(1, 4).astype(f32)
    ctxT = jnp.transpose(ctx_coords, (0, 2, 1))
    main_args = [
        ctxT, ctx_values, tgt_coords,
        jnp.asarray(_LHSA_NP), jnp.asarray(_RHSB_NP), scal,
        jnp.asarray(_G_NP),
    ] + list(w_exp)
    inv_k = 1.0 / K5
    for l, lp in enumerate(params["cnn"]):
        if l < 2:
            main_args.append(inv_k * jnp.transpose(lp["W"]))
        else:
            co, ci = _CHOUT[l], _CHIN[l]
            w2 = jnp.transpose(lp["W"].reshape(co, _CMCO, ci), (2, 1, 0))
            main_args.append(inv_k * w2.reshape(ci, _CMCO * co))
    for l, lp in enumerate(params["cnn"]):
        main_args.append(inv_k * lp["b"].reshape(1, _CHOUT[l]))
    for l in range(4):
        main_args.append(jnp.asarray(_T_NP[l]))

    def batched(shape):
        return pl.BlockSpec((1,) + shape, lambda b: (b,) + (0,) * len(shape))

    def full(arr):
        nd = arr.ndim
        return pl.BlockSpec(arr.shape, lambda b, _n=nd: (0,) * _n)

    in_specs = [
        batched((2, NC)), batched((NC, 1)), batched((NT, 2)),
    ] + [full(a) for a in main_args[3:]]

    res, diag = pl.pallas_call(
        _main_body,
        grid=(B,),
        in_specs=in_specs,
        out_specs=[batched((NT, 2)), batched((NT, NT))],
        out_shape=[jax.ShapeDtypeStruct((B, NT, 2), f32),
                   jax.ShapeDtypeStruct((B, NT, NT), f32)],
        interpret=_INTERPRET,
    )(*main_args)

    return res[..., 0], diag
